# baseline (speedup 1.0000x reference)
import functools

import jax
import jax.numpy as jnp
from jax import lax
from jax.experimental import pallas as pl
from jax.experimental.pallas import tpu as pltpu
from jax.experimental.pallas import tpu_sc as plsc

ROW_TILE = 128
SC_CHUNK = 48


def _router_body(h_ref, wr_ref, w_ref, e_ref):
    h = h_ref[...]
    wr = wr_ref[...]
    logits = jnp.dot(h, wr, preferred_element_type=jnp.float32)
    s, e = logits.shape
    col = lax.broadcasted_iota(jnp.int32, (s, e), 1)
    a1 = jnp.argmax(logits, axis=1).astype(jnp.int32)
    m1 = jnp.max(logits, axis=1)
    masked = jnp.where(col == a1[:, None], -jnp.inf, logits)
    a2 = jnp.argmax(masked, axis=1).astype(jnp.int32)
    m2 = jnp.max(masked, axis=1)
    t = jnp.exp(m2 - m1)
    wa = 1.0 / (1.0 + t)
    wb = 1.0 - wa
    w_ref[...] = jnp.concatenate([wa[:, None], wb[:, None]], axis=1)
    e_ref[...] = jnp.concatenate([a1[:, None], a2[:, None]], axis=1)


def _sc_gather(token_map, h2, n_pad):
    s, d_model = h2.shape
    info = plsc.get_sparse_core_info()
    nw = info.num_cores * info.num_subcores
    b_per_w = n_pad // nw
    n_chunks = b_per_w // SC_CHUNK
    mesh = plsc.VectorSubcoreMesh(core_axis_name="c", subcore_axis_name="s")

    @functools.partial(
        pl.kernel,
        mesh=mesh,
        out_type=jax.ShapeDtypeStruct((n_pad, d_model), jnp.float32),
        scratch_types=[
            pltpu.VMEM((b_per_w,), jnp.int32),
            pltpu.VMEM((SC_CHUNK, d_model), jnp.float32),
            pltpu.VMEM((SC_CHUNK, d_model), jnp.float32),
            pltpu.SemaphoreType.DMA,
            pltpu.SemaphoreType.DMA,
        ],
    )
    def gather_k(tm_hbm, h_hbm, x_hbm, idx_v, rows0, rows1, sem0, sem1):
        wid = lax.axis_index("s") * info.num_cores + lax.axis_index("c")
        base = wid * b_per_w
        pltpu.sync_copy(tm_hbm.at[pl.ds(base, b_per_w)], idx_v)
        bufs = (rows0, rows1)
        sems = (sem0, sem1)
        copies = [None] * n_chunks
        copies[0] = pltpu.async_copy(
            h_hbm.at[idx_v.at[pl.ds(0, SC_CHUNK)]], bufs[0], sems[0])
        for c in range(n_chunks):
            if c + 1 < n_chunks:
                copies[c + 1] = pltpu.async_copy(
                    h_hbm.at[idx_v.at[pl.ds((c + 1) * SC_CHUNK, SC_CHUNK)]],
                    bufs[(c + 1) % 2], sems[(c + 1) % 2])
            copies[c].wait()
            pltpu.sync_copy(bufs[c % 2],
                            x_hbm.at[pl.ds(base + c * SC_CHUNK, SC_CHUNK)])

    return gather_k(token_map, h2)


def _ffn_body(te_ref, tm_ref, h_ref, w1_ref, v1_ref, w2_ref, y_ref, x_scr):
    i = pl.program_id(0)
    expert = te_ref[i]

    @pl.when(expert >= 0)
    def _():
        def gather_row(r, carry):
            tok = tm_ref[i * ROW_TILE + r]
            x_scr[pl.ds(r, 1), :] = h_ref[pl.ds(tok, 1), :]
            return carry

        lax.fori_loop(0, ROW_TILE, gather_row, 0)
        x = x_scr[...]
        t1 = jnp.dot(x, w1_ref[0], preferred_element_type=jnp.float32)
        t2 = jnp.dot(x, v1_ref[0], preferred_element_type=jnp.float32)
        g = t1 * jax.nn.sigmoid(t1) * t2
        y_ref[...] = jnp.dot(g, w2_ref[0], preferred_element_type=jnp.float32)


def _combine_body(pa_ref, pb_ref, wab_ref, y_ref, out_ref, ya_scr, yb_scr):
    i = pl.program_id(0)

    def gather_row(r, carry):
        pa = pa_ref[i * ROW_TILE + r]
        pb = pb_ref[i * ROW_TILE + r]
        ya_scr[pl.ds(r, 1), :] = y_ref[pl.ds(pa, 1), :]
        yb_scr[pl.ds(r, 1), :] = y_ref[pl.ds(pb, 1), :]
        return carry

    lax.fori_loop(0, ROW_TILE, gather_row, 0)
    wa = wab_ref[:, 0:1]
    wb = wab_ref[:, 1:2]
    out_ref[...] = wa * ya_scr[...] + wb * yb_scr[...]


def kernel(hidden_states, w_router, w1, v1, w2):
    batch, seq, d_model = hidden_states.shape
    n_experts, _, ffn = w1.shape
    s = batch * seq
    top_k = 2
    n_assign = s * top_k
    n_tiles = n_assign // ROW_TILE + n_experts
    n_pad = n_tiles * ROW_TILE

    h2 = hidden_states.reshape(s, d_model)

    wab, eab = pl.pallas_call(
        _router_body,
        out_shape=(
            jax.ShapeDtypeStruct((s, top_k), jnp.float32),
            jax.ShapeDtypeStruct((s, top_k), jnp.int32),
        ),
    )(h2, w_router)

    e_flat = eab.reshape(-1)
    onehot = (e_flat[:, None] == jnp.arange(n_experts)[None, :]).astype(jnp.int32)
    cum = jnp.cumsum(onehot, axis=0)
    counts = cum[-1]
    rank = jnp.take_along_axis(cum, e_flat[:, None], axis=1)[:, 0] - 1
    tiles_per = (counts + ROW_TILE - 1) // ROW_TILE
    tile_start = jnp.concatenate([jnp.zeros((1,), jnp.int32),
                                  jnp.cumsum(tiles_per)[:-1].astype(jnp.int32)])
    pstart = tile_start * ROW_TILE
    slot = pstart[e_flat] + rank
    token_map = jnp.zeros((n_pad,), jnp.int32).at[slot].set(
        (jnp.arange(n_assign, dtype=jnp.int32) // top_k))
    total_tiles = tile_start[-1] + tiles_per[-1]
    tile_ids = jnp.arange(n_tiles, dtype=jnp.int32)
    tile_expert = jnp.searchsorted(tile_start, tile_ids, side="right").astype(jnp.int32) - 1
    tile_expert = jnp.where(tile_ids < total_tiles, tile_expert, -1)
    slot2 = slot.reshape(s, top_k)
    pa, pb = slot2[:, 0], slot2[:, 1]

    grid_spec = pltpu.PrefetchScalarGridSpec(
        num_scalar_prefetch=2,
        grid=(n_tiles,),
        in_specs=[
            pl.BlockSpec((s, d_model), lambda i, te, tm: (0, 0)),
            pl.BlockSpec((1, d_model, ffn),
                         lambda i, te, tm: (jnp.maximum(te[i], 0), 0, 0)),
            pl.BlockSpec((1, d_model, ffn),
                         lambda i, te, tm: (jnp.maximum(te[i], 0), 0, 0)),
            pl.BlockSpec((1, ffn, d_model),
                         lambda i, te, tm: (jnp.maximum(te[i], 0), 0, 0)),
        ],
        out_specs=pl.BlockSpec((ROW_TILE, d_model), lambda i, te, tm: (i, 0)),
        scratch_shapes=[pltpu.VMEM((ROW_TILE, d_model), jnp.float32)],
    )
    y = pl.pallas_call(
        _ffn_body,
        grid_spec=grid_spec,
        out_shape=jax.ShapeDtypeStruct((n_pad, d_model), jnp.float32),
        compiler_params=pltpu.CompilerParams(
            vmem_limit_bytes=100 * 1024 * 1024),
    )(tile_expert, token_map, h2, w1, v1, w2)
    return y

    comb_spec = pltpu.PrefetchScalarGridSpec(
        num_scalar_prefetch=2,
        grid=(s // ROW_TILE,),
        in_specs=[
            pl.BlockSpec((ROW_TILE, top_k), lambda i, pa_, pb_: (i, 0)),
            pl.BlockSpec((n_pad, d_model), lambda i, pa_, pb_: (0, 0)),
        ],
        out_specs=pl.BlockSpec((ROW_TILE, d_model), lambda i, pa_, pb_: (i, 0)),
        scratch_shapes=[
            pltpu.VMEM((ROW_TILE, d_model), jnp.float32),
            pltpu.VMEM((ROW_TILE, d_model), jnp.float32),
        ],
    )
    out = pl.pallas_call(
        _combine_body,
        grid_spec=comb_spec,
        out_shape=jax.ShapeDtypeStruct((s, d_model), jnp.float32),
        compiler_params=pltpu.CompilerParams(
            vmem_limit_bytes=100 * 1024 * 1024),
    )(pa, pb, wab, y)

    return out.reshape(batch, seq, d_model)


# speedup vs baseline: 1.0109x; 1.0109x over previous
import jax
import jax.numpy as jnp
from jax import lax
from jax.experimental import pallas as pl
from jax.experimental.pallas import tpu as pltpu

ROW_TILE = 128


def _router_body(h_ref, wr_ref, wab_ref, slots_ref, te_ref):
    h = h_ref[...]
    logits = jnp.dot(h, wr_ref[...], preferred_element_type=jnp.float32)
    s, e = logits.shape
    n_tiles = te_ref.shape[0]
    col = lax.broadcasted_iota(jnp.int32, (s, e), 1)
    a1 = jnp.argmax(logits, axis=1).astype(jnp.int32)
    m1 = jnp.max(logits, axis=1)
    masked = jnp.where(col == a1[:, None], -jnp.inf, logits)
    a2 = jnp.argmax(masked, axis=1).astype(jnp.int32)
    m2 = jnp.max(masked, axis=1)
    t = jnp.exp(m2 - m1)
    wa = 1.0 / (1.0 + t)
    wb = 1.0 - wa
    wab_ref[...] = jnp.concatenate([wa[:, None], wb[:, None]], axis=1)

    oh1 = (col == a1[:, None]).astype(jnp.float32)
    oh2 = (col == a2[:, None]).astype(jnp.float32)
    st = oh1 + oh2
    ri = lax.broadcasted_iota(jnp.int32, (ROW_TILE, ROW_TILE), 0)
    ci = lax.broadcasted_iota(jnp.int32, (ROW_TILE, ROW_TILE), 1)
    lt = (ci < ri).astype(jnp.float32)
    off = jnp.zeros((1, e), jnp.float32)
    parts = []
    for b in range(s // ROW_TILE):
        blk = st[b * ROW_TILE:(b + 1) * ROW_TILE, :]
        pin = jnp.dot(lt, blk, preferred_element_type=jnp.float32)
        parts.append(pin + off)
        off = off + jnp.sum(blk, axis=0, keepdims=True)
    p = jnp.concatenate(parts, axis=0)
    counts = off
    rank1 = jnp.sum(p * oh1, axis=1)
    rank2 = jnp.sum(p * oh2, axis=1)
    tiles_per = jnp.floor((counts + (ROW_TILE - 1)) * (1.0 / ROW_TILE))
    r16 = lax.broadcasted_iota(jnp.int32, (e, e), 0)
    c16 = lax.broadcasted_iota(jnp.int32, (e, e), 1)
    m16 = (r16 < c16).astype(jnp.float32)
    ts = jnp.dot(tiles_per, m16, preferred_element_type=jnp.float32)
    pstart = ts * float(ROW_TILE)
    slot1 = jnp.sum(oh1 * pstart, axis=1) + rank1
    slot2 = jnp.sum(oh2 * pstart, axis=1) + rank2
    slots_ref[...] = jnp.concatenate(
        [slot1[:, None], slot2[:, None]], axis=1).astype(jnp.int32)
    tid = lax.broadcasted_iota(jnp.int32, (n_tiles, e), 0)
    ts_i = ts.astype(jnp.int32)
    ge = (tid >= ts_i).astype(jnp.int32)
    te = jnp.sum(ge, axis=1) - 1
    oh_last = (lax.broadcasted_iota(jnp.int32, (1, e), 1) == (e - 1))
    total = jnp.sum(jnp.where(oh_last, ts + tiles_per, 0.0), axis=1,
                    keepdims=True).astype(jnp.int32)
    te = jnp.where(tid[:, 0:1] < total, te[:, None], -1)
    te_ref[...] = te


def _ffn_body(te_ref, tm_ref, h_ref, w1_ref, v1_ref, w2_ref, y_ref, x_scr):
    i = pl.program_id(0)
    expert = te_ref[i]

    @pl.when(expert >= 0)
    def _():
        def gather_row(r, carry):
            tok = tm_ref[i * ROW_TILE + r]
            x_scr[pl.ds(r, 1), :] = h_ref[pl.ds(tok, 1), :]
            return carry

        lax.fori_loop(0, ROW_TILE, gather_row, 0)
        x = x_scr[...]
        t1 = jnp.dot(x, w1_ref[0], preferred_element_type=jnp.float32)
        t2 = jnp.dot(x, v1_ref[0], preferred_element_type=jnp.float32)
        g = t1 * jax.nn.sigmoid(t1) * t2
        y_ref[...] = jnp.dot(g, w2_ref[0], preferred_element_type=jnp.float32)


def _combine_body(pa_ref, pb_ref, wab_ref, y_ref, out_ref, ya_scr, yb_scr):
    i = pl.program_id(0)

    def gather_row(r, carry):
        pa = pa_ref[i * ROW_TILE + r]
        pb = pb_ref[i * ROW_TILE + r]
        ya_scr[pl.ds(r, 1), :] = y_ref[pl.ds(pa, 1), :]
        yb_scr[pl.ds(r, 1), :] = y_ref[pl.ds(pb, 1), :]
        return carry

    lax.fori_loop(0, ROW_TILE, gather_row, 0)
    wa = wab_ref[:, 0:1]
    wb = wab_ref[:, 1:2]
    out_ref[...] = wa * ya_scr[...] + wb * yb_scr[...]


def kernel(hidden_states, w_router, w1, v1, w2):
    batch, seq, d_model = hidden_states.shape
    n_experts, _, ffn = w1.shape
    s = batch * seq
    top_k = 2
    n_assign = s * top_k
    n_tiles = n_assign // ROW_TILE + n_experts - 1
    n_pad = n_tiles * ROW_TILE

    h2 = hidden_states.reshape(s, d_model)

    wab, slots, te = pl.pallas_call(
        _router_body,
        out_shape=(
            jax.ShapeDtypeStruct((s, top_k), jnp.float32),
            jax.ShapeDtypeStruct((s, top_k), jnp.int32),
            jax.ShapeDtypeStruct((n_tiles, 1), jnp.int32),
        ),
    )(h2, w_router)

    token_map = jnp.zeros((n_pad,), jnp.int32).at[slots.reshape(-1)].set(
        jnp.arange(n_assign, dtype=jnp.int32) // top_k,
        mode="promise_in_bounds", unique_indices=True)
    tile_expert = te.reshape(-1)
    pa, pb = slots[:, 0], slots[:, 1]

    grid_spec = pltpu.PrefetchScalarGridSpec(
        num_scalar_prefetch=2,
        grid=(n_tiles,),
        in_specs=[
            pl.BlockSpec((s, d_model), lambda i, te_, tm: (0, 0)),
            pl.BlockSpec((1, d_model, ffn),
                         lambda i, te_, tm: (jnp.maximum(te_[i], 0), 0, 0)),
            pl.BlockSpec((1, d_model, ffn),
                         lambda i, te_, tm: (jnp.maximum(te_[i], 0), 0, 0)),
            pl.BlockSpec((1, ffn, d_model),
                         lambda i, te_, tm: (jnp.maximum(te_[i], 0), 0, 0)),
        ],
        out_specs=pl.BlockSpec((ROW_TILE, d_model), lambda i, te_, tm: (i, 0)),
        scratch_shapes=[pltpu.VMEM((ROW_TILE, d_model), jnp.float32)],
    )
    y = pl.pallas_call(
        _ffn_body,
        grid_spec=grid_spec,
        out_shape=jax.ShapeDtypeStruct((n_pad, d_model), jnp.float32),
        compiler_params=pltpu.CompilerParams(
            vmem_limit_bytes=100 * 1024 * 1024),
    )(tile_expert, token_map, h2, w1, v1, w2)

    comb_spec = pltpu.PrefetchScalarGridSpec(
        num_scalar_prefetch=2,
        grid=(s // ROW_TILE,),
        in_specs=[
            pl.BlockSpec((ROW_TILE, top_k), lambda i, pa_, pb_: (i, 0)),
            pl.BlockSpec((n_pad, d_model), lambda i, pa_, pb_: (0, 0)),
        ],
        out_specs=pl.BlockSpec((ROW_TILE, d_model), lambda i, pa_, pb_: (i, 0)),
        scratch_shapes=[
            pltpu.VMEM((ROW_TILE, d_model), jnp.float32),
            pltpu.VMEM((ROW_TILE, d_model), jnp.float32),
        ],
    )
    out = pl.pallas_call(
        _combine_body,
        grid_spec=comb_spec,
        out_shape=jax.ShapeDtypeStruct((s, d_model), jnp.float32),
        compiler_params=pltpu.CompilerParams(
            vmem_limit_bytes=100 * 1024 * 1024),
    )(pa, pb, wab, y)

    return out.reshape(batch, seq, d_model)
